# baseline (speedup 1.0000x reference)
import functools

import jax
import jax.numpy as jnp
from jax import lax
from jax.experimental import pallas as pl
from jax.experimental.pallas import tpu as pltpu
from jax.experimental.pallas import tpu_sc as plsc

N_NODES = 10000
N_EDGES = 320000
D_FEAT = 128
D_HIDDEN = 256

NC = 2
NS = 16
NW = NC * NS
CHUNK = 128
N_CHUNKS_PAD = 2560
MY_CHUNKS = N_CHUNKS_PAD // NW
N_PAD = 10240
ROWS_PER_TILE = N_PAD // NS
ZROWS = 128


def _sc_body(feat_hbm, src_hbm, dst_hbm, acc_out, deg_out,
             src_a, dst_a, src_b, dst_b, rows0, rows1,
             deg_local, acc_shared, sem_a, sem_b):
  c = lax.axis_index("c")
  s = lax.axis_index("s")
  wid = s * NC + c

  zeros16 = jnp.zeros((16,), jnp.float32)

  def zero_deg(j, _):
    deg_local[pl.ds(j * 16, 16)] = zeros16
    return 0

  lax.fori_loop(0, N_PAD // 16, zero_deg, 0)

  def zero_rows(j, _):
    for k in range(D_FEAT // 16):
      rows0[j, pl.ds(k * 16, 16)] = zeros16
    return 0

  lax.fori_loop(0, ZROWS, zero_rows, 0)

  row0 = s * ROWS_PER_TILE
  for k in range(ROWS_PER_TILE // ZROWS):
    pltpu.sync_copy(rows0, acc_shared.at[pl.ds(row0 + k * ZROWS, ZROWS)])
  plsc.subcore_barrier()

  ones16 = jnp.ones((16,), jnp.float32)

  def base_of(i):
    return (wid + i * NW) * CHUNK

  def idx_sync(i, sbuf, dbuf):
    pltpu.sync_copy(src_hbm.at[pl.ds(base_of(i), CHUNK)], sbuf)
    pltpu.sync_copy(dst_hbm.at[pl.ds(base_of(i), CHUNK)], dbuf)

  def deg_update(dbuf):
    for j in range(CHUNK // 16):
      d16 = dbuf[pl.ds(j * 16, 16)]
      plsc.addupdate_scatter(deg_local, [d16], ones16)

  def pair(q, last):
    d0 = pltpu.async_copy(feat_hbm.at[src_a], rows0, sem_a)
    idx_sync(2 * q + 1, src_b, dst_b)
    d1 = pltpu.async_copy(feat_hbm.at[src_b], rows1, sem_b)
    d0.wait()
    deg_update(dst_a)
    pltpu.sync_copy(rows0, acc_shared.at[dst_a], add=True)
    if not last:
      idx_sync(2 * q + 2, src_a, dst_a)
    d1.wait()
    deg_update(dst_b)
    pltpu.sync_copy(rows1, acc_shared.at[dst_b], add=True)

  idx_sync(0, src_a, dst_a)

  def q_body(q, _):
    pair(q, last=False)
    return 0

  lax.fori_loop(0, MY_CHUNKS // 2 - 1, q_body, 0)
  pair(MY_CHUNKS // 2 - 1, last=True)
  plsc.subcore_barrier()

  pltpu.sync_copy(deg_local, deg_out.at[wid, 0])
  for k in range(ROWS_PER_TILE // ZROWS):
    r = row0 + k * ZROWS
    pltpu.sync_copy(acc_shared.at[pl.ds(r, ZROWS)], acc_out.at[c, pl.ds(r, ZROWS)])


@jax.jit
def _sc_aggregate(features, src, dst):
  mesh = plsc.VectorSubcoreMesh(core_axis_name="c", subcore_axis_name="s")
  return pl.kernel(
      _sc_body,
      out_type=[
          jax.ShapeDtypeStruct((NC, N_PAD, D_FEAT), jnp.float32),
          jax.ShapeDtypeStruct((NW, 1, N_PAD), jnp.float32),
      ],
      mesh=mesh,
      compiler_params=pltpu.CompilerParams(needs_layout_passes=False),
      scratch_types=[
          pltpu.VMEM((CHUNK,), jnp.int32),
          pltpu.VMEM((CHUNK,), jnp.int32),
          pltpu.VMEM((CHUNK,), jnp.int32),
          pltpu.VMEM((CHUNK,), jnp.int32),
          pltpu.VMEM((CHUNK, D_FEAT), jnp.float32),
          pltpu.VMEM((CHUNK, D_FEAT), jnp.float32),
          pltpu.VMEM((N_PAD,), jnp.float32),
          pltpu.VMEM_SHARED((N_PAD, D_FEAT), jnp.float32),
          pltpu.SemaphoreType.DMA,
          pltpu.SemaphoreType.DMA,
      ],
  )(features, src, dst)


def _tc_head_body(acc_ref, deg_ref, wmp_ref, bmp_ref, w1_ref, b1_ref,
                  w2_ref, b2_ref, out_ref):
  acc = acc_ref[0] + acc_ref[1]
  deg = jnp.sum(deg_ref[0], axis=0)
  deg = jnp.maximum(deg, 1.0)
  h = acc / deg[:, None]
  h = jnp.maximum(jnp.dot(h, wmp_ref[...], preferred_element_type=jnp.float32)
                  + bmp_ref[...], 0.0)
  h = jnp.maximum(jnp.dot(h, w1_ref[...], preferred_element_type=jnp.float32)
                  + b1_ref[...], 0.0)
  out_ref[...] = (jnp.dot(h, w2_ref[...], preferred_element_type=jnp.float32)
                  + b2_ref[...])


@jax.jit
def _tc_head(acc2, deg32, W_mp, b_mp, W1, b1, W2, b2):
  R = 1000
  grid = (N_NODES // R,)
  f = pl.pallas_call(
      _tc_head_body,
      grid=grid,
      in_specs=[
          pl.BlockSpec((NC, R, D_FEAT), lambda i: (0, i, 0)),
          pl.BlockSpec((1, NW, R), lambda i: (i, 0, 0)),
          pl.BlockSpec((D_FEAT, D_FEAT), lambda i: (0, 0)),
          pl.BlockSpec((1, D_FEAT), lambda i: (0, 0)),
          pl.BlockSpec((D_FEAT, D_HIDDEN), lambda i: (0, 0)),
          pl.BlockSpec((1, D_HIDDEN), lambda i: (0, 0)),
          pl.BlockSpec((D_HIDDEN, D_FEAT), lambda i: (0, 0)),
          pl.BlockSpec((1, D_FEAT), lambda i: (0, 0)),
      ],
      out_specs=pl.BlockSpec((R, D_FEAT), lambda i: (i, 0)),
      out_shape=jax.ShapeDtypeStruct((N_NODES, D_FEAT), jnp.float32),
  )
  deg_t = deg32.reshape(NW, N_NODES // R, R).transpose(1, 0, 2)
  return f(acc2, deg_t, W_mp, b_mp, W1, b1, W2, b2)


@jax.jit
def _full(features, edge_index, W_mp, b_mp, W1, b1, W2, b2):
  src = edge_index[0].astype(jnp.int32)
  dst = edge_index[1].astype(jnp.int32)
  n_extra = N_CHUNKS_PAD * CHUNK - N_EDGES
  src = jnp.concatenate([src, jnp.zeros((n_extra,), jnp.int32)])
  dst = jnp.concatenate([dst, jnp.full((n_extra,), N_PAD - 1, jnp.int32)])
  acc2, deg32 = _sc_aggregate(features, src, dst)
  deg32 = deg32.reshape(NW, N_PAD)[:, :N_NODES]
  return _tc_head(acc2, deg32, W_mp, b_mp.reshape(1, -1),
                  W1, b1.reshape(1, -1), W2, b2.reshape(1, -1))


def kernel(features, edge_index, W_mp, b_mp, W1, b1, W2, b2):
  return _full(features, edge_index, W_mp, b_mp, W1, b1, W2, b2)


# speedup vs baseline: 1.8050x; 1.8050x over previous
import functools

import jax
import jax.numpy as jnp
from jax import lax
from jax.experimental import pallas as pl
from jax.experimental.pallas import tpu as pltpu
from jax.experimental.pallas import tpu_sc as plsc

N_NODES = 10000
N_EDGES = 320000
D_FEAT = 128
D_HIDDEN = 256

NC = 2
NS = 16
NW = NC * NS
CHUNK = 128
N_CHUNKS = N_EDGES // CHUNK
N_PAD = 10240
ROWS_PER_TILE = N_PAD // NS
ZROWS = 128


def _sc_body(feat_hbm, edges_hbm, acc_out, deg_out,
             ebuf, dst_idx, rows, zbuf, deg_local, acc_shared, sem):
  c = lax.axis_index("c")
  s = lax.axis_index("s")
  wid = s * NC + c

  zeros16 = jnp.zeros((16,), jnp.float32)

  def zero_deg(j, _):
    deg_local[pl.ds(j * 16, 16)] = zeros16
    return 0

  lax.fori_loop(0, N_NODES // 16, zero_deg, 0)

  def zero_zbuf(j, _):
    for k in range(D_FEAT // 16):
      zbuf[j, pl.ds(k * 16, 16)] = zeros16
    return 0

  lax.fori_loop(0, ZROWS, zero_zbuf, 0)

  row0 = s * ROWS_PER_TILE
  for k in range(ROWS_PER_TILE // ZROWS):
    pltpu.sync_copy(zbuf, acc_shared.at[pl.ds(row0 + k * ZROWS, ZROWS)])
  plsc.subcore_barrier()

  ones16 = jnp.ones((16,), jnp.float32)
  n_my_chunks = (N_CHUNKS - 1 - wid) // NW + 1

  def edge_step(i, _):
    base = (wid + i * NW) * (2 * CHUNK)
    pltpu.sync_copy(edges_hbm.at[pl.ds(base, 2 * CHUNK)], ebuf)
    d = pltpu.async_copy(feat_hbm.at[ebuf.at[pl.ds(0, CHUNK)]], rows, sem)
    for j in range(CHUNK // 16):
      dst_idx[pl.ds(j * 16, 16)] = ebuf[pl.ds(CHUNK + j * 16, 16)]
    d.wait()
    pltpu.sync_copy(rows, acc_shared.at[dst_idx], add=True)
    for j in range(CHUNK // 16):
      d16 = dst_idx[pl.ds(j * 16, 16)]
      plsc.addupdate_scatter(deg_local, [d16], ones16)
    return 0

  lax.fori_loop(0, n_my_chunks, edge_step, 0)
  plsc.subcore_barrier()

  pltpu.sync_copy(deg_local, deg_out.at[wid, 0])
  for k in range(ROWS_PER_TILE // ZROWS):
    r = row0 + k * ZROWS
    pltpu.sync_copy(acc_shared.at[pl.ds(r, ZROWS)], acc_out.at[c, pl.ds(r, ZROWS)])


@jax.jit
def _sc_aggregate(features, edges):
  mesh = plsc.VectorSubcoreMesh(core_axis_name="c", subcore_axis_name="s")
  return pl.kernel(
      _sc_body,
      out_type=[
          jax.ShapeDtypeStruct((NC, N_PAD, D_FEAT), jnp.float32),
          jax.ShapeDtypeStruct((NW, 1, N_NODES), jnp.float32),
      ],
      mesh=mesh,
      compiler_params=pltpu.CompilerParams(needs_layout_passes=False),
      scratch_types=[
          pltpu.VMEM((2 * CHUNK,), jnp.int32),
          pltpu.VMEM((CHUNK,), jnp.int32),
          pltpu.VMEM((CHUNK, D_FEAT), jnp.float32),
          pltpu.VMEM((ZROWS, D_FEAT), jnp.float32),
          pltpu.VMEM((N_NODES,), jnp.float32),
          pltpu.VMEM_SHARED((N_PAD, D_FEAT), jnp.float32),
          pltpu.SemaphoreType.DMA,
      ],
  )(features, edges)


def _tc_head_body(acc_ref, deg_ref, wmp_ref, bmp_ref, w1_ref, b1_ref,
                  w2_ref, b2_ref, out_ref):
  acc = acc_ref[0] + acc_ref[1]
  deg = jnp.sum(deg_ref[0], axis=0)
  deg = jnp.maximum(deg, 1.0)
  h = acc / deg[:, None]
  h = jnp.maximum(jnp.dot(h, wmp_ref[...], preferred_element_type=jnp.float32)
                  + bmp_ref[...], 0.0)
  h = jnp.maximum(jnp.dot(h, w1_ref[...], preferred_element_type=jnp.float32)
                  + b1_ref[...], 0.0)
  out_ref[...] = (jnp.dot(h, w2_ref[...], preferred_element_type=jnp.float32)
                  + b2_ref[...])


@jax.jit
def _tc_head(acc2, deg32, W_mp, b_mp, W1, b1, W2, b2):
  R = 1000
  grid = (N_NODES // R,)
  f = pl.pallas_call(
      _tc_head_body,
      grid=grid,
      in_specs=[
          pl.BlockSpec((NC, R, D_FEAT), lambda i: (0, i, 0)),
          pl.BlockSpec((1, NW, R), lambda i: (i, 0, 0)),
          pl.BlockSpec((D_FEAT, D_FEAT), lambda i: (0, 0)),
          pl.BlockSpec((1, D_FEAT), lambda i: (0, 0)),
          pl.BlockSpec((D_FEAT, D_HIDDEN), lambda i: (0, 0)),
          pl.BlockSpec((1, D_HIDDEN), lambda i: (0, 0)),
          pl.BlockSpec((D_HIDDEN, D_FEAT), lambda i: (0, 0)),
          pl.BlockSpec((1, D_FEAT), lambda i: (0, 0)),
      ],
      out_specs=pl.BlockSpec((R, D_FEAT), lambda i: (i, 0)),
      out_shape=jax.ShapeDtypeStruct((N_NODES, D_FEAT), jnp.float32),
  )
  deg_t = deg32.reshape(NW, N_NODES // R, R).transpose(1, 0, 2)
  return f(acc2, deg_t, W_mp, b_mp, W1, b1, W2, b2)


@jax.jit
def _full(features, edge_index, W_mp, b_mp, W1, b1, W2, b2):
  src = edge_index[0].astype(jnp.int32)
  dst = edge_index[1].astype(jnp.int32)
  edges = jnp.stack([src.reshape(N_CHUNKS, CHUNK),
                     dst.reshape(N_CHUNKS, CHUNK)], axis=1).reshape(-1)
  acc2, deg32 = _sc_aggregate(features, edges)
  deg32 = deg32.reshape(NW, N_NODES)
  return _tc_head(acc2, deg32, W_mp, b_mp.reshape(1, -1),
                  W1, b1.reshape(1, -1), W2, b2.reshape(1, -1))


def kernel(features, edge_index, W_mp, b_mp, W1, b1, W2, b2):
  return _full(features, edge_index, W_mp, b_mp, W1, b1, W2, b2)


# speedup vs baseline: 1.8349x; 1.0165x over previous
import functools

import jax
import jax.numpy as jnp
from jax import lax
from jax.experimental import pallas as pl
from jax.experimental.pallas import tpu as pltpu
from jax.experimental.pallas import tpu_sc as plsc

N_NODES = 10000
N_EDGES = 320000
D_FEAT = 128
D_HIDDEN = 256

NC = 2
NS = 16
NW = NC * NS
CHUNK = 128
N_CHUNKS = N_EDGES // CHUNK
N_PAD = 10240
ROWS_PER_TILE = N_PAD // NS
ZROWS = 128


def _sc_body(feat_hbm, edges_hbm, acc_out, deg_out,
             ebuf, dst_idx, rows, zbuf, deg_local, acc_shared, sem):
  c = lax.axis_index("c")
  s = lax.axis_index("s")
  wid = s * NC + c

  zeros16 = jnp.zeros((16,), jnp.float32)

  def zero_deg(j, _):
    deg_local[pl.ds(j * 16, 16)] = zeros16
    return 0

  lax.fori_loop(0, N_NODES // 16, zero_deg, 0)

  def zero_zbuf(j, _):
    for k in range(D_FEAT // 16):
      zbuf[j, pl.ds(k * 16, 16)] = zeros16
    return 0

  lax.fori_loop(0, ZROWS, zero_zbuf, 0)

  row0 = s * ROWS_PER_TILE
  for k in range(ROWS_PER_TILE // ZROWS):
    pltpu.sync_copy(zbuf, acc_shared.at[pl.ds(row0 + k * ZROWS, ZROWS)])
  plsc.subcore_barrier()

  ones16 = jnp.ones((16,), jnp.float32)
  n_my_chunks = (N_CHUNKS - 1 - wid) // NW + 1

  def edge_step(i, _):
    base = (wid + i * NW) * (2 * CHUNK)
    pltpu.sync_copy(edges_hbm.at[pl.ds(base, 2 * CHUNK)], ebuf)
    d = pltpu.async_copy(feat_hbm.at[ebuf.at[pl.ds(0, CHUNK)]], rows, sem)
    for j in range(CHUNK // 16):
      d16 = ebuf[pl.ds(CHUNK + j * 16, 16)]
      dst_idx[pl.ds(j * 16, 16)] = d16
      plsc.addupdate_scatter(deg_local, [d16], ones16)
    d.wait()
    pltpu.sync_copy(rows, acc_shared.at[dst_idx], add=True)
    return 0

  lax.fori_loop(0, n_my_chunks, edge_step, 0)
  plsc.subcore_barrier()

  pltpu.sync_copy(deg_local, deg_out.at[wid, 0])
  for k in range(ROWS_PER_TILE // ZROWS):
    r = row0 + k * ZROWS
    pltpu.sync_copy(acc_shared.at[pl.ds(r, ZROWS)], acc_out.at[c, pl.ds(r, ZROWS)])


@jax.jit
def _sc_aggregate(features, edges):
  mesh = plsc.VectorSubcoreMesh(core_axis_name="c", subcore_axis_name="s")
  return pl.kernel(
      _sc_body,
      out_type=[
          jax.ShapeDtypeStruct((NC, N_PAD, D_FEAT), jnp.float32),
          jax.ShapeDtypeStruct((NW, 1, N_NODES), jnp.float32),
      ],
      mesh=mesh,
      compiler_params=pltpu.CompilerParams(needs_layout_passes=False),
      scratch_types=[
          pltpu.VMEM((2 * CHUNK,), jnp.int32),
          pltpu.VMEM((CHUNK,), jnp.int32),
          pltpu.VMEM((CHUNK, D_FEAT), jnp.float32),
          pltpu.VMEM((ZROWS, D_FEAT), jnp.float32),
          pltpu.VMEM((N_NODES,), jnp.float32),
          pltpu.VMEM_SHARED((N_PAD, D_FEAT), jnp.float32),
          pltpu.SemaphoreType.DMA,
      ],
  )(features, edges)


def _tc_head_body(acc_ref, deg_ref, wmp_ref, bmp_ref, w1_ref, b1_ref,
                  w2_ref, b2_ref, out_ref):
  acc = acc_ref[0] + acc_ref[1]
  deg = jnp.sum(deg_ref[0], axis=0)
  deg = jnp.maximum(deg, 1.0)
  h = acc / deg[:, None]
  h = jnp.maximum(jnp.dot(h, wmp_ref[...], preferred_element_type=jnp.float32)
                  + bmp_ref[...], 0.0)
  h = jnp.maximum(jnp.dot(h, w1_ref[...], preferred_element_type=jnp.float32)
                  + b1_ref[...], 0.0)
  out_ref[...] = (jnp.dot(h, w2_ref[...], preferred_element_type=jnp.float32)
                  + b2_ref[...])


@jax.jit
def _tc_head(acc2, deg32, W_mp, b_mp, W1, b1, W2, b2):
  R = 1000
  grid = (N_NODES // R,)
  f = pl.pallas_call(
      _tc_head_body,
      grid=grid,
      in_specs=[
          pl.BlockSpec((NC, R, D_FEAT), lambda i: (0, i, 0)),
          pl.BlockSpec((1, NW, R), lambda i: (i, 0, 0)),
          pl.BlockSpec((D_FEAT, D_FEAT), lambda i: (0, 0)),
          pl.BlockSpec((1, D_FEAT), lambda i: (0, 0)),
          pl.BlockSpec((D_FEAT, D_HIDDEN), lambda i: (0, 0)),
          pl.BlockSpec((1, D_HIDDEN), lambda i: (0, 0)),
          pl.BlockSpec((D_HIDDEN, D_FEAT), lambda i: (0, 0)),
          pl.BlockSpec((1, D_FEAT), lambda i: (0, 0)),
      ],
      out_specs=pl.BlockSpec((R, D_FEAT), lambda i: (i, 0)),
      out_shape=jax.ShapeDtypeStruct((N_NODES, D_FEAT), jnp.float32),
  )
  deg_t = deg32.reshape(NW, N_NODES // R, R).transpose(1, 0, 2)
  return f(acc2, deg_t, W_mp, b_mp, W1, b1, W2, b2)


@jax.jit
def _full(features, edge_index, W_mp, b_mp, W1, b1, W2, b2):
  src = edge_index[0].astype(jnp.int32)
  dst = edge_index[1].astype(jnp.int32)
  edges = jnp.stack([src.reshape(N_CHUNKS, CHUNK),
                     dst.reshape(N_CHUNKS, CHUNK)], axis=1).reshape(-1)
  acc2, deg32 = _sc_aggregate(features, edges)
  deg32 = deg32.reshape(NW, N_NODES)
  return _tc_head(acc2, deg32, W_mp, b_mp.reshape(1, -1),
                  W1, b1.reshape(1, -1), W2, b2.reshape(1, -1))


def kernel(features, edge_index, W_mp, b_mp, W1, b1, W2, b2):
  return _full(features, edge_index, W_mp, b_mp, W1, b1, W2, b2)
